# baseline (speedup 1.0000x reference)
import functools

import jax
import jax.numpy as jnp
from jax import lax
from jax.experimental import pallas as pl
from jax.experimental.pallas import tpu as pltpu
from jax.experimental.pallas import tpu_sc as plsc

B, L, H = 1024, 200, 128
N = B * L
NC, NS, LANES = 2, 16, 16
NW = NC * NS
TPW = N // NW
C = 32
NCHUNK = TPW // C
GROUPS = C // LANES
JJ = H // LANES
HP = H // 2
PJ = HP // LANES
EPS = 1e-6


def _tec_body(ids_hbm, bbox_hbm, word_hbm, postt_hbm, x_hbm, y_hbm, h_hbm,
              w_hbm, lnw_hbm, lnb_hbm, out_hbm,
              ids0, ids1, bbox0, bbox1, xi_v, yi_v, hi_v, wi_v,
              wbuf0, wbuf1, postt_v, xbuf0, xbuf1, ybuf0, ybuf1,
              hbuf0, hbuf1, qbuf0, qbuf1, lnw_v, lnb_v,
              semg, semst, semout):
    wid = lax.axis_index("s") * NC + lax.axis_index("c")
    tbase = wid * TPW
    pltpu.sync_copy(lnw_hbm, lnw_v)
    pltpu.sync_copy(lnb_hbm, lnb_v)
    pltpu.sync_copy(postt_hbm, postt_v)
    iota = lax.iota(jnp.int32, LANES)

    ids = (ids0, ids1)
    bbox = (bbox0, bbox1)
    wbuf = (wbuf0, wbuf1)
    xbuf = (xbuf0, xbuf1)
    ybuf = (ybuf0, ybuf1)
    hbuf = (hbuf0, hbuf1)
    qbuf = (qbuf0, qbuf1)

    def stage(k, s):
        sbase = lax.min(tbase + k * C, N - C)
        pltpu.async_copy(ids_hbm.at[pl.ds(sbase, C)], ids[s], semst.at[s])
        pltpu.async_copy(bbox_hbm.at[pl.ds(sbase * 4, 4 * C)], bbox[s],
                         semst.at[s])

    def wait_stage(s):
        pltpu.make_async_copy(ids_hbm.at[pl.ds(0, C)], ids[s],
                              semst.at[s]).wait()
        pltpu.make_async_copy(bbox_hbm.at[pl.ds(0, 4 * C)], bbox[s],
                              semst.at[s]).wait()

    def build_idx(s):
        for g in range(GROUPS):
            p4 = iota * 4 + (4 * LANES * g)
            b0 = plsc.load_gather(bbox[s], [p4])
            b1 = plsc.load_gather(bbox[s], [p4 + 1])
            b2 = plsc.load_gather(bbox[s], [p4 + 2])
            b3 = plsc.load_gather(bbox[s], [p4 + 3])
            sl = pl.ds(g * LANES, LANES)
            sh = pl.ds(C + g * LANES, LANES)
            xi_v[sl] = b0
            xi_v[sh] = b2
            yi_v[sl] = b1
            yi_v[sh] = b3
            hi_v[sl] = b3 - b1
            wi_v[sl] = b2 - b0

    def issue_gathers(k, s):
        pltpu.async_copy(word_hbm.at[ids[s]], wbuf[s], semg.at[s])
        pltpu.async_copy(x_hbm.at[xi_v], xbuf[s], semg.at[s])
        pltpu.async_copy(y_hbm.at[yi_v], ybuf[s], semg.at[s])
        pltpu.async_copy(h_hbm.at[hi_v], hbuf[s], semg.at[s])
        pltpu.async_copy(w_hbm.at[wi_v], qbuf[s], semg.at[s])

    def wait_gathers(s):
        pltpu.make_async_copy(word_hbm.at[ids[s]], wbuf[s], semg.at[s]).wait()
        pltpu.make_async_copy(x_hbm.at[xi_v], xbuf[s], semg.at[s]).wait()
        pltpu.make_async_copy(y_hbm.at[yi_v], ybuf[s], semg.at[s]).wait()
        pltpu.make_async_copy(h_hbm.at[hi_v], hbuf[s], semg.at[s]).wait()
        pltpu.make_async_copy(w_hbm.at[wi_v], qbuf[s], semg.at[s]).wait()

    def wait_out(s):
        pltpu.make_async_copy(wbuf[s], out_hbm.at[pl.ds(0, C)],
                              semout.at[s]).wait()

    MASK_HI = jnp.int32(-65536)

    def compute(s, p0, lnw, lnb):
        wb, xb, yb, hb, qb = (wbuf[s], xbuf[s], ybuf[s], hbuf[s], qbuf[s])

        def row(r):
            vs = []
            for j in range(JJ):
                sj = pl.ds(j * LANES, LANES)
                v = ((wb[r, sj] + xb[r, sj]) + (xb[C + r, sj] + yb[r, sj])
                     + (yb[C + r, sj] + hb[r, sj]) + qb[r, sj])
                vs.append(v)
            for j2 in range(PJ):
                xi32 = postt_v[p0 + r, pl.ds(j2 * LANES, LANES)]
                lo = lax.bitcast_convert_type(
                    lax.shift_left(xi32, 16), jnp.float32)
                hi = lax.bitcast_convert_type(
                    lax.bitwise_and(xi32, MASK_HI), jnp.float32)
                vs[2 * j2] = vs[2 * j2] + lo
                vs[2 * j2 + 1] = vs[2 * j2 + 1] + hi
            s8 = ((vs[0] + vs[1]) + (vs[2] + vs[3])) + \
                 ((vs[4] + vs[5]) + (vs[6] + vs[7]))
            q = [v * v for v in vs]
            q8 = ((q[0] + q[1]) + (q[2] + q[3])) + \
                 ((q[4] + q[5]) + (q[6] + q[7]))
            mean = jnp.sum(s8) * (1.0 / H)
            ex2 = jnp.sum(q8) * (1.0 / H)
            var = ex2 - mean * mean + EPS
            iv = lax.bitcast_convert_type(var, jnp.int32)
            iv = jnp.int32(0x5F3759DF) - lax.shift_right_logical(iv, 1)
            y = lax.bitcast_convert_type(iv, jnp.float32)
            for _ in range(3):
                y = y * (1.5 - 0.5 * var * y * y)
            for j in range(JJ):
                wb[r, pl.ds(j * LANES, LANES)] = \
                    (vs[j] - mean) * y * lnw[j] + lnb[j]

        def body(r2, carry):
            row(2 * r2)
            row(2 * r2 + 1)
            return carry

        lax.fori_loop(0, C // 2, body, (lnw, lnb))

    stage(0, 0)
    wait_stage(0)
    build_idx(0)
    issue_gathers(0, 0)
    stage(1, 1)

    lnw = tuple(lnw_v[pl.ds(j * LANES, LANES)] for j in range(JJ))
    lnb = tuple(lnb_v[pl.ds(j * LANES, LANES)] for j in range(JJ))

    def half_iter(k, s, sn, first, lnw, lnb):
        base = tbase + k * C
        wait_gathers(s)
        wait_stage(sn)
        build_idx(sn)
        if first:
            @pl.when(k > 0)
            def _():
                wait_out(sn)
        else:
            wait_out(sn)
        issue_gathers(k + 1, sn)
        stage(k + 2, s)
        compute(s, lax.rem(k * C, L), lnw, lnb)
        pltpu.async_copy(wbuf[s], out_hbm.at[pl.ds(base, C)], semout.at[s])

    def loop_body(k2, carry):
        lnw, lnb = carry
        k = 2 * k2
        half_iter(k, 0, 1, True, lnw, lnb)
        half_iter(k + 1, 1, 0, False, lnw, lnb)
        return carry

    lax.fori_loop(0, NCHUNK // 2, loop_body, (lnw, lnb))

    wait_gathers(0)
    wait_stage(1)
    wait_out(1)


@jax.jit
def _run(ids_flat, bbox_flat, word_embeddings, postt_p, x_p, y_p, h_p, w_p,
         ln_weight, ln_bias):
    mesh = plsc.VectorSubcoreMesh(core_axis_name="c", subcore_axis_name="s")
    f = functools.partial(
        pl.kernel,
        out_type=jax.ShapeDtypeStruct((N, H), jnp.float32),
        mesh=mesh,
        scratch_types=[
            pltpu.VMEM((C,), jnp.int32),
            pltpu.VMEM((C,), jnp.int32),
            pltpu.VMEM((4 * C,), jnp.int32),
            pltpu.VMEM((4 * C,), jnp.int32),
            pltpu.VMEM((2 * C,), jnp.int32),
            pltpu.VMEM((2 * C,), jnp.int32),
            pltpu.VMEM((C,), jnp.int32),
            pltpu.VMEM((C,), jnp.int32),
            pltpu.VMEM((C, H), jnp.float32),
            pltpu.VMEM((C, H), jnp.float32),
            pltpu.VMEM((L + C, HP), jnp.int32),
            pltpu.VMEM((2 * C, H), jnp.float32),
            pltpu.VMEM((2 * C, H), jnp.float32),
            pltpu.VMEM((2 * C, H), jnp.float32),
            pltpu.VMEM((2 * C, H), jnp.float32),
            pltpu.VMEM((C, H), jnp.float32),
            pltpu.VMEM((C, H), jnp.float32),
            pltpu.VMEM((C, H), jnp.float32),
            pltpu.VMEM((C, H), jnp.float32),
            pltpu.VMEM((H,), jnp.float32),
            pltpu.VMEM((H,), jnp.float32),
            pltpu.SemaphoreType.DMA((2,)),
            pltpu.SemaphoreType.DMA((2,)),
            pltpu.SemaphoreType.DMA((2,)),
        ],
        compiler_params=pltpu.CompilerParams(needs_layout_passes=False),
    )(_tec_body)
    return f(ids_flat, bbox_flat, word_embeddings, postt_p, x_p, y_p, h_p,
             w_p, ln_weight, ln_bias)


def _pack_bf16_pairs(t):
    v = t.shape[0]
    tr = t.reshape(v, PJ, 2, LANES)
    lob = lax.bitcast_convert_type(
        tr[:, :, 0, :].astype(jnp.bfloat16), jnp.uint16).astype(jnp.uint32)
    hib = lax.bitcast_convert_type(
        tr[:, :, 1, :].astype(jnp.bfloat16), jnp.uint16).astype(jnp.uint32)
    return lax.bitcast_convert_type(
        (lob | (hib << 16)).reshape(v, HP), jnp.int32)


def kernel(input_ids, bbox, word_embeddings, position_embeddings,
           token_type_embeddings, x_position_embeddings,
           y_position_embeddings, h_position_embeddings,
           w_position_embeddings, ln_weight, ln_bias):
    ids_flat = input_ids.reshape(-1)
    bbox_flat = bbox.reshape(-1)
    postt = position_embeddings + token_type_embeddings[0][None, :]
    postt_ext = jnp.concatenate([postt[:L], postt[:C]], axis=0)
    out = _run(ids_flat, bbox_flat, word_embeddings,
               _pack_bf16_pairs(postt_ext),
               x_position_embeddings, y_position_embeddings,
               h_position_embeddings, w_position_embeddings,
               ln_weight, ln_bias)
    return out.reshape(B, L, H)


# speedup vs baseline: 1.0530x; 1.0530x over previous
import functools

import jax
import jax.numpy as jnp
from jax import lax
from jax.experimental import pallas as pl
from jax.experimental.pallas import tpu as pltpu
from jax.experimental.pallas import tpu_sc as plsc

B, L, H = 1024, 200, 128
N = B * L
NC, NS, LANES = 2, 16, 16
NW = NC * NS
TPW = N // NW
C = 32
NCHUNK = TPW // C
GROUPS = C // LANES
JJ = H // LANES
HP = H // 2
PJ = HP // LANES
EPS = 1e-6


def _tec_body(ids_hbm, bbox_hbm, word_hbm, postt_hbm, x_hbm, y_hbm, h_hbm,
              w_hbm, lnw_hbm, lnb_hbm, out_hbm,
              ids0, ids1, bbox0, bbox1, xi_v, yi_v, hi_v, wi_v,
              wbuf0, wbuf1, pbuf0, pbuf1, xbuf0, xbuf1, ybuf0, ybuf1,
              hbuf0, hbuf1, qbuf0, qbuf1, lnw_v, lnb_v,
              semg, semst, semout):
    wid = lax.axis_index("s") * NC + lax.axis_index("c")
    tbase = wid * TPW
    pltpu.sync_copy(lnw_hbm, lnw_v)
    pltpu.sync_copy(lnb_hbm, lnb_v)
    iota = lax.iota(jnp.int32, LANES)

    ids = (ids0, ids1)
    bbox = (bbox0, bbox1)
    wbuf = (wbuf0, wbuf1)
    pbuf = (pbuf0, pbuf1)
    xbuf = (xbuf0, xbuf1)
    ybuf = (ybuf0, ybuf1)
    hbuf = (hbuf0, hbuf1)
    qbuf = (qbuf0, qbuf1)

    def stage(k, s):
        sbase = lax.min(tbase + k * C, N - C)
        pltpu.async_copy(ids_hbm.at[pl.ds(sbase, C)], ids[s], semst.at[s])
        pltpu.async_copy(bbox_hbm.at[pl.ds(sbase * 4, 4 * C)], bbox[s],
                         semst.at[s])

    def wait_stage(s):
        pltpu.make_async_copy(ids_hbm.at[pl.ds(0, C)], ids[s],
                              semst.at[s]).wait()
        pltpu.make_async_copy(bbox_hbm.at[pl.ds(0, 4 * C)], bbox[s],
                              semst.at[s]).wait()

    def build_idx(s):
        for g in range(GROUPS):
            p4 = iota * 4 + (4 * LANES * g)
            b0 = plsc.load_gather(bbox[s], [p4])
            b1 = plsc.load_gather(bbox[s], [p4 + 1])
            b2 = plsc.load_gather(bbox[s], [p4 + 2])
            b3 = plsc.load_gather(bbox[s], [p4 + 3])
            sl = pl.ds(g * LANES, LANES)
            sh = pl.ds(C + g * LANES, LANES)
            xi_v[sl] = b0
            xi_v[sh] = b2
            yi_v[sl] = b1
            yi_v[sh] = b3
            hi_v[sl] = b3 - b1
            wi_v[sl] = b2 - b0

    def issue_gathers(k, s):
        p0 = lax.rem(k * C, L)
        pltpu.async_copy(word_hbm.at[ids[s]], wbuf[s], semg.at[s])
        pltpu.async_copy(postt_hbm.at[pl.ds(p0, C)], pbuf[s], semg.at[s])
        pltpu.async_copy(x_hbm.at[xi_v], xbuf[s], semg.at[s])
        pltpu.async_copy(y_hbm.at[yi_v], ybuf[s], semg.at[s])
        pltpu.async_copy(h_hbm.at[hi_v], hbuf[s], semg.at[s])
        pltpu.async_copy(w_hbm.at[wi_v], qbuf[s], semg.at[s])

    def wait_gathers(s):
        pltpu.make_async_copy(word_hbm.at[ids[s]], wbuf[s], semg.at[s]).wait()
        pltpu.make_async_copy(postt_hbm.at[pl.ds(0, C)], pbuf[s],
                              semg.at[s]).wait()
        pltpu.make_async_copy(x_hbm.at[xi_v], xbuf[s], semg.at[s]).wait()
        pltpu.make_async_copy(y_hbm.at[yi_v], ybuf[s], semg.at[s]).wait()
        pltpu.make_async_copy(h_hbm.at[hi_v], hbuf[s], semg.at[s]).wait()
        pltpu.make_async_copy(w_hbm.at[wi_v], qbuf[s], semg.at[s]).wait()

    def wait_out(s):
        pltpu.make_async_copy(wbuf[s], out_hbm.at[pl.ds(0, C)],
                              semout.at[s]).wait()

    MASK_HI = jnp.int32(-65536)

    def compute(s, lnw, lnb):
        wb, pb, xb, yb, hb, qb = (wbuf[s], pbuf[s], xbuf[s], ybuf[s],
                                  hbuf[s], qbuf[s])

        def row(r):
            vs = []
            for j in range(JJ):
                sj = pl.ds(j * LANES, LANES)
                v = ((wb[r, sj] + xb[r, sj]) + (xb[C + r, sj] + yb[r, sj])
                     + (yb[C + r, sj] + hb[r, sj]) + qb[r, sj])
                vs.append(v)
            for j2 in range(PJ):
                xi32 = pb[r, pl.ds(j2 * LANES, LANES)]
                lo = lax.bitcast_convert_type(
                    lax.shift_left(xi32, 16), jnp.float32)
                hi = lax.bitcast_convert_type(
                    lax.bitwise_and(xi32, MASK_HI), jnp.float32)
                vs[2 * j2] = vs[2 * j2] + lo
                vs[2 * j2 + 1] = vs[2 * j2 + 1] + hi
            s8 = ((vs[0] + vs[1]) + (vs[2] + vs[3])) + \
                 ((vs[4] + vs[5]) + (vs[6] + vs[7]))
            q = [v * v for v in vs]
            q8 = ((q[0] + q[1]) + (q[2] + q[3])) + \
                 ((q[4] + q[5]) + (q[6] + q[7]))
            mean = jnp.sum(s8) * (1.0 / H)
            ex2 = jnp.sum(q8) * (1.0 / H)
            var = ex2 - mean * mean + EPS
            iv = lax.bitcast_convert_type(var, jnp.int32)
            iv = jnp.int32(0x5F3759DF) - lax.shift_right_logical(iv, 1)
            y = lax.bitcast_convert_type(iv, jnp.float32)
            for _ in range(3):
                y = y * (1.5 - 0.5 * var * y * y)
            for j in range(JJ):
                wb[r, pl.ds(j * LANES, LANES)] = \
                    (vs[j] - mean) * y * lnw[j] + lnb[j]

        def body(r2, carry):
            row(2 * r2)
            row(2 * r2 + 1)
            return carry

        lax.fori_loop(0, C // 2, body, (lnw, lnb))

    stage(0, 0)
    wait_stage(0)
    build_idx(0)
    issue_gathers(0, 0)
    stage(1, 1)

    lnw = tuple(lnw_v[pl.ds(j * LANES, LANES)] for j in range(JJ))
    lnb = tuple(lnb_v[pl.ds(j * LANES, LANES)] for j in range(JJ))

    def half_iter(k, s, sn, first, lnw, lnb):
        base = tbase + k * C
        wait_gathers(s)
        wait_stage(sn)
        build_idx(sn)
        if first:
            @pl.when(k > 0)
            def _():
                wait_out(sn)
        else:
            wait_out(sn)
        issue_gathers(k + 1, sn)
        stage(k + 2, s)
        compute(s, lnw, lnb)
        pltpu.async_copy(wbuf[s], out_hbm.at[pl.ds(base, C)], semout.at[s])

    def loop_body(k2, carry):
        lnw, lnb = carry
        k = 2 * k2
        half_iter(k, 0, 1, True, lnw, lnb)
        half_iter(k + 1, 1, 0, False, lnw, lnb)
        return carry

    lax.fori_loop(0, NCHUNK // 2, loop_body, (lnw, lnb))

    wait_gathers(0)
    wait_stage(1)
    wait_out(1)


@jax.jit
def _run(ids_flat, bbox_flat, word_embeddings, postt_p, x_p, y_p, h_p, w_p,
         ln_weight, ln_bias):
    mesh = plsc.VectorSubcoreMesh(core_axis_name="c", subcore_axis_name="s")
    f = functools.partial(
        pl.kernel,
        out_type=jax.ShapeDtypeStruct((N, H), jnp.float32),
        mesh=mesh,
        scratch_types=[
            pltpu.VMEM((C,), jnp.int32),
            pltpu.VMEM((C,), jnp.int32),
            pltpu.VMEM((4 * C,), jnp.int32),
            pltpu.VMEM((4 * C,), jnp.int32),
            pltpu.VMEM((2 * C,), jnp.int32),
            pltpu.VMEM((2 * C,), jnp.int32),
            pltpu.VMEM((C,), jnp.int32),
            pltpu.VMEM((C,), jnp.int32),
            pltpu.VMEM((C, H), jnp.float32),
            pltpu.VMEM((C, H), jnp.float32),
            pltpu.VMEM((C, HP), jnp.int32),
            pltpu.VMEM((C, HP), jnp.int32),
            pltpu.VMEM((2 * C, H), jnp.float32),
            pltpu.VMEM((2 * C, H), jnp.float32),
            pltpu.VMEM((2 * C, H), jnp.float32),
            pltpu.VMEM((2 * C, H), jnp.float32),
            pltpu.VMEM((C, H), jnp.float32),
            pltpu.VMEM((C, H), jnp.float32),
            pltpu.VMEM((C, H), jnp.float32),
            pltpu.VMEM((C, H), jnp.float32),
            pltpu.VMEM((H,), jnp.float32),
            pltpu.VMEM((H,), jnp.float32),
            pltpu.SemaphoreType.DMA((2,)),
            pltpu.SemaphoreType.DMA((2,)),
            pltpu.SemaphoreType.DMA((2,)),
        ],
        compiler_params=pltpu.CompilerParams(needs_layout_passes=False),
    )(_tec_body)
    return f(ids_flat, bbox_flat, word_embeddings, postt_p, x_p, y_p, h_p,
             w_p, ln_weight, ln_bias)


def _pack_bf16_pairs(t):
    v = t.shape[0]
    tr = t.reshape(v, PJ, 2, LANES)
    lob = lax.bitcast_convert_type(
        tr[:, :, 0, :].astype(jnp.bfloat16), jnp.uint16).astype(jnp.uint32)
    hib = lax.bitcast_convert_type(
        tr[:, :, 1, :].astype(jnp.bfloat16), jnp.uint16).astype(jnp.uint32)
    return lax.bitcast_convert_type(
        (lob | (hib << 16)).reshape(v, HP), jnp.int32)


def kernel(input_ids, bbox, word_embeddings, position_embeddings,
           token_type_embeddings, x_position_embeddings,
           y_position_embeddings, h_position_embeddings,
           w_position_embeddings, ln_weight, ln_bias):
    ids_flat = input_ids.reshape(-1)
    bbox_flat = bbox.reshape(-1)
    postt = position_embeddings + token_type_embeddings[0][None, :]
    postt_ext = jnp.concatenate([postt[:L], postt[:C]], axis=0)
    out = _run(ids_flat, bbox_flat, word_embeddings,
               _pack_bf16_pairs(postt_ext),
               x_position_embeddings, y_position_embeddings,
               h_position_embeddings, w_position_embeddings,
               ln_weight, ln_bias)
    return out.reshape(B, L, H)
